# baseline (speedup 1.0000x reference)
import functools

import jax
import jax.numpy as jnp
from jax import lax
from jax.experimental import pallas as pl
from jax.experimental.pallas import tpu as pltpu
from jax.experimental.pallas import tpu_sc as plsc

D_MODEL = 1024
SCALE = 32.0
NC, NS, L = 2, 16, 16
NW = NC * NS


@functools.lru_cache(maxsize=None)
def _make_emb(B: int):
    assert B % NW == 0
    b_per_w = B // NW
    chunk = 32
    assert b_per_w % chunk == 0
    n_chunks = b_per_w // chunk

    mesh = plsc.VectorSubcoreMesh(
        core_axis_name="c", subcore_axis_name="s",
        num_cores=NC, num_subcores=NS)

    @functools.partial(
        pl.kernel,
        out_type=jax.ShapeDtypeStruct((B, D_MODEL), jnp.float32),
        mesh=mesh,
        scratch_types=[
            pltpu.VMEM((b_per_w,), jnp.int32),
            pltpu.VMEM((chunk, D_MODEL), jnp.float32),
            pltpu.SemaphoreType.DMA,
        ],
    )
    def _emb(idx_hbm, table_hbm, out_hbm, idx_v, buf, sem):
        wid = lax.axis_index("s") * NC + lax.axis_index("c")
        base = wid * b_per_w
        pltpu.sync_copy(idx_hbm.at[pl.ds(base, b_per_w)], idx_v)

        @pl.loop(0, n_chunks)
        def _chunk(g):
            pltpu.async_copy(
                table_hbm.at[idx_v.at[pl.ds(g * chunk, chunk)]], buf, sem
            ).wait()

            @pl.loop(0, chunk)
            def _row(r):
                for i in range(D_MODEL // L):
                    buf[r, pl.ds(i * L, L)] = buf[r, pl.ds(i * L, L)] * SCALE

            pltpu.sync_copy(buf, out_hbm.at[pl.ds(base + g * chunk, chunk)])

    return _emb


def kernel(x, table):
    idx = x.reshape(-1).astype(jnp.int32)
    out = _make_emb(idx.shape[0])(idx, table)
    return out.reshape(x.shape + (D_MODEL,))


# speedup vs baseline: 1.1638x; 1.1638x over previous
import functools

import jax
import jax.numpy as jnp
from jax import lax
from jax.experimental import pallas as pl
from jax.experimental.pallas import tpu as pltpu
from jax.experimental.pallas import tpu_sc as plsc

D_MODEL = 1024
SCALE = 32.0
NC, NS, L = 2, 16, 16
NW = NC * NS


@functools.lru_cache(maxsize=None)
def _make_emb(B: int):
    assert B % NW == 0
    b_per_w = B // NW
    chunk = 16
    nbuf = 2
    assert b_per_w % chunk == 0
    n_chunks = b_per_w // chunk
    assert n_chunks % nbuf == 0

    mesh = plsc.VectorSubcoreMesh(
        core_axis_name="c", subcore_axis_name="s",
        num_cores=NC, num_subcores=NS)

    @functools.partial(
        pl.kernel,
        out_type=jax.ShapeDtypeStruct((B, D_MODEL), jnp.float32),
        mesh=mesh,
        scratch_types=[
            pltpu.VMEM((b_per_w,), jnp.int32),
            pltpu.VMEM((nbuf, chunk, D_MODEL), jnp.float32),
            pltpu.VMEM((nbuf, chunk, D_MODEL), jnp.float32),
            [pltpu.SemaphoreType.DMA] * nbuf,
            [pltpu.SemaphoreType.DMA] * nbuf,
        ],
    )
    def _emb(idx_hbm, table_hbm, out_hbm, idx_v, ibuf, obuf, gsems, ssems):
        wid = lax.axis_index("s") * NC + lax.axis_index("c")
        base = wid * b_per_w
        pltpu.sync_copy(idx_hbm.at[pl.ds(base, b_per_w)], idx_v)

        def gather(g, b):
            return pltpu.make_async_copy(
                table_hbm.at[idx_v.at[pl.ds(g * chunk, chunk)]],
                ibuf.at[b], gsems[b])

        def scatter(g, b):
            return pltpu.make_async_copy(
                obuf.at[b], out_hbm.at[pl.ds(base + g * chunk, chunk)],
                ssems[b])

        for b in range(nbuf):
            gather(b, b).start()

        @pl.loop(0, n_chunks, step=nbuf)
        def _outer(g0):
            for b in range(nbuf):
                g = g0 + b
                gather(g, b).wait()

                @pl.when(g0 > 0)
                def _():
                    scatter(g - nbuf, b).wait()

                @pl.loop(0, chunk)
                def _row(r):
                    for i in range(D_MODEL // L):
                        obuf[b, r, pl.ds(i * L, L)] = (
                            ibuf[b, r, pl.ds(i * L, L)] * SCALE)

                @pl.when(g0 + nbuf < n_chunks)
                def _():
                    gather(g + nbuf, b).start()

                scatter(g, b).start()

        for b in range(nbuf):
            scatter(n_chunks - nbuf + b, b).wait()

    return _emb


def kernel(x, table):
    idx = x.reshape(-1).astype(jnp.int32)
    out = _make_emb(idx.shape[0])(idx, table)
    return out.reshape(x.shape + (D_MODEL,))


# speedup vs baseline: 1.6552x; 1.4223x over previous
import functools

import jax
import jax.numpy as jnp
from jax import lax
from jax.experimental import pallas as pl
from jax.experimental.pallas import tpu as pltpu
from jax.experimental.pallas import tpu_sc as plsc

D_MODEL = 1024
SCALE = 32.0
NC, NS, L = 2, 16, 16
NW = NC * NS


@functools.lru_cache(maxsize=None)
def _make_emb(B: int):
    assert B % NW == 0
    b_per_w = B // NW
    chunk = 8
    nbuf = 4
    assert b_per_w % chunk == 0
    n_chunks = b_per_w // chunk
    assert n_chunks % nbuf == 0

    mesh = plsc.VectorSubcoreMesh(
        core_axis_name="c", subcore_axis_name="s",
        num_cores=NC, num_subcores=NS)

    @functools.partial(
        pl.kernel,
        out_type=jax.ShapeDtypeStruct((B, D_MODEL), jnp.float32),
        mesh=mesh,
        scratch_types=[
            pltpu.VMEM((b_per_w,), jnp.int32),
            pltpu.VMEM((nbuf, chunk, D_MODEL), jnp.float32),
            pltpu.VMEM((nbuf, chunk, D_MODEL), jnp.float32),
            [pltpu.SemaphoreType.DMA] * nbuf,
            [pltpu.SemaphoreType.DMA] * nbuf,
        ],
    )
    def _emb(idx_hbm, table_hbm, out_hbm, idx_v, ibuf, obuf, gsems, ssems):
        wid = lax.axis_index("s") * NC + lax.axis_index("c")
        base = wid * b_per_w
        pltpu.sync_copy(idx_hbm.at[pl.ds(base, b_per_w)], idx_v)

        def gather(g, b):
            return pltpu.make_async_copy(
                table_hbm.at[idx_v.at[pl.ds(g * chunk, chunk)]],
                ibuf.at[b], gsems[b])

        def scatter(g, b):
            return pltpu.make_async_copy(
                obuf.at[b], out_hbm.at[pl.ds(base + g * chunk, chunk)],
                ssems[b])

        for b in range(nbuf):
            gather(b, b).start()

        @pl.loop(0, n_chunks, step=nbuf)
        def _outer(g0):
            for b in range(nbuf):
                g = g0 + b
                gather(g, b).wait()

                @pl.when(g0 > 0)
                def _():
                    scatter(g - nbuf, b).wait()

                @pl.loop(0, chunk)
                def _row(r):
                    for i in range(D_MODEL // L):
                        obuf[b, r, pl.ds(i * L, L)] = (
                            ibuf[b, r, pl.ds(i * L, L)] * SCALE)

                @pl.when(g0 + nbuf < n_chunks)
                def _():
                    gather(g + nbuf, b).start()

                scatter(g, b).start()

        for b in range(nbuf):
            scatter(n_chunks - nbuf + b, b).wait()

    return _emb


def kernel(x, table):
    idx = x.reshape(-1).astype(jnp.int32)
    out = _make_emb(idx.shape[0])(idx, table)
    return out.reshape(x.shape + (D_MODEL,))
